# baseline (speedup 1.0000x reference)
import jax
import jax.numpy as jnp
from jax import lax
from jax.experimental import pallas as pl
from jax.experimental.pallas import tpu as pltpu
from jax.experimental.pallas import tpu_sc as plsc

NC = 2
NS = 16
NW = NC * NS
S = 200
BB = 128
V = 1000000
CW = 512
VT = V // CW
VTAIL = V - VT * CW


def _relayout_body(tT_hbm, tail_hbm, out_hbm, in_v, tout_v,
                   isem0, isem1, osem0, osem1):
    wid = lax.axis_index("s") * NC + lax.axis_index("c")
    iota = lax.iota(jnp.int32, 16)

    def transpose(p, nrow):
        def row(r4, carry):
            col0 = 4 * r4
            vecs = []
            for j in range(8):
                rid = iota + 16 * (j % 2)
                cid = jnp.full((16,), 0, jnp.int32) + (col0 + j // 2)
                vecs.append(plsc.load_gather(in_v.at[p], [rid, cid]))
            for j in range(8):
                tout_v[p, r4, pl.ds(16 * j, 16)] = vecs[j]
            return carry
        lax.fori_loop(0, nrow, row, 0)

    def in_copy(m, p, sem):
        rt = m * NW + wid
        return pltpu.make_async_copy(
            tT_hbm.at[:, pl.ds(rt * CW, CW)], in_v.at[p], sem)

    def out_copy(m, p, sem):
        rt = m * NW + wid
        return pltpu.make_async_copy(
            tout_v.at[p], out_hbm.at[pl.ds(rt * (CW // 4), CW // 4)], sem)

    def valid(m):
        return (m * NW + wid) < VT

    M = 2 * ((VT + 2 * NW - 1) // (2 * NW))

    @pl.when(valid(0))
    def _():
        in_copy(0, 0, isem0).start()

    @pl.when(valid(1))
    def _():
        in_copy(1, 1, isem1).start()

    def pair(j2, carry):
        for p, isem, osem in ((0, isem0, osem0), (1, isem1, osem1)):
            m = 2 * j2 + p

            @pl.when(valid(m))
            def _():
                in_copy(m, p, isem).wait()

            @pl.when((m >= 2) & valid(m - 2))
            def _():
                out_copy(m - 2, p, osem).wait()

            @pl.when(valid(m))
            def _():
                transpose(p, CW // 4)

            @pl.when(valid(m + 2))
            def _():
                in_copy(m + 2, p, isem).start()

            @pl.when(valid(m))
            def _():
                out_copy(m, p, osem).start()
        return carry

    lax.fori_loop(0, M // 2, pair, 0)

    @pl.when(valid(M - 2))
    def _():
        out_copy(M - 2, 0, osem0).wait()

    @pl.when(valid(M - 1))
    def _():
        out_copy(M - 1, 1, osem1).wait()

    @pl.when(wid == 0)
    def _():
        pltpu.sync_copy(tail_hbm, in_v.at[0, :, pl.ds(0, BB)])
        transpose(0, 16)
        pltpu.sync_copy(tout_v.at[0, pl.ds(0, 16)],
                        out_hbm.at[pl.ds(VT * (CW // 4), 16)])


def _gather_body(idxT_hbm, table_hbm, out_hbm, idx_v, rows_v, tout_v,
                 gsem0, gsem1, ssem0, ssem1):
    wid = lax.axis_index("s") * NC + lax.axis_index("c")
    pltpu.sync_copy(idxT_hbm.at[:, pl.ds(wid * BB, BB)], idx_v)
    iota = lax.iota(jnp.int32, 16)

    def gather(s, p, sem):
        return pltpu.make_async_copy(
            table_hbm.at[idx_v.at[s]], rows_v.at[p], sem)

    def store(s, p, sem):
        return pltpu.make_async_copy(
            tout_v.at[p], out_hbm.at[s, :, pl.ds(wid * 1024, 1024)], sem)

    def transpose(p):
        for bk in range(8):
            rid = iota + bk * 16
            for fg in range(4):
                vecs = [
                    plsc.load_gather(
                        rows_v.at[p],
                        [rid, jnp.full((16,), fg * 8 + fr, jnp.int32)])
                    for fr in range(8)
                ]
                for fr in range(8):
                    tout_v[p, fg, pl.ds(fr * BB + bk * 16, 16)] = vecs[fr]

    gather(0, 0, gsem0).start()
    gather(1, 1, gsem1).start()

    def pair(j2, carry):
        for p, gsem, ssem in ((0, gsem0, ssem0), (1, gsem1, ssem1)):
            s = 2 * j2 + p
            gather(s, p, gsem).wait()

            @pl.when(j2 >= 1)
            def _():
                store(s - 2, p, ssem).wait()

            transpose(p)

            @pl.when(j2 < S // 2 - 1)
            def _():
                gather(s + 2, p, gsem).start()

            store(s, p, ssem).start()
        return carry

    lax.fori_loop(0, S // 2, pair, 0)
    store(S - 2, 0, ssem0).wait()
    store(S - 1, 1, ssem1).wait()


def kernel(inputs, embedding):
    bt, s = inputs.shape
    v, d = embedding.shape
    assert (bt, s, v, d) == (NW * BB, S, V, 32)

    mesh = plsc.VectorSubcoreMesh(core_axis_name="c", subcore_axis_name="s")

    tT = embedding.T
    tail = jnp.pad(tT[:, VT * CW:], ((0, 0), (0, BB - VTAIL)))
    k1 = pl.kernel(
        _relayout_body,
        out_type=jax.ShapeDtypeStruct((V * d // BB, BB), jnp.float32),
        mesh=mesh,
        scratch_types=[
            pltpu.VMEM((2, d, CW), jnp.float32),
            pltpu.VMEM((2, CW // 4, BB), jnp.float32),
            pltpu.SemaphoreType.DMA,
            pltpu.SemaphoreType.DMA,
            pltpu.SemaphoreType.DMA,
            pltpu.SemaphoreType.DMA,
        ],
        compiler_params=pltpu.CompilerParams(
            use_tc_tiling_on_sc=True, needs_layout_passes=False),
    )
    table_rm = k1(tT, tail).reshape(v, d)

    idxT = inputs.T.astype(jnp.int32)
    k2 = pl.kernel(
        _gather_body,
        out_type=jax.ShapeDtypeStruct((S, 4, 8 * BB * NW), jnp.float32),
        mesh=mesh,
        scratch_types=[
            pltpu.VMEM((S, BB), jnp.int32),
            pltpu.VMEM((2, BB, d), jnp.float32),
            pltpu.VMEM((2, 4, 8 * BB), jnp.float32),
            pltpu.SemaphoreType.DMA,
            pltpu.SemaphoreType.DMA,
            pltpu.SemaphoreType.DMA,
            pltpu.SemaphoreType.DMA,
        ],
        compiler_params=pltpu.CompilerParams(
            use_tc_tiling_on_sc=False, needs_layout_passes=False),
    )
    out5 = k2(idxT, table_rm)
    return (out5.reshape(S, 4, NW, 8, BB)
                .transpose(2, 4, 0, 1, 3)
                .reshape(bt, s, d))


# speedup vs baseline: 1.0156x; 1.0156x over previous
import jax
import jax.numpy as jnp
from jax import lax
from jax.experimental import pallas as pl
from jax.experimental.pallas import tpu as pltpu
from jax.experimental.pallas import tpu_sc as plsc

NC = 2
NS = 16
NW = NC * NS
S = 200
BB = 128
V = 1000000
CW = 512
VT = V // CW
VTAIL = V - VT * CW


def _relayout_body(tT_hbm, tail_hbm, out_hbm, in_v, tout_v,
                   isem0, isem1, osem0, osem1):
    wid = lax.axis_index("s") * NC + lax.axis_index("c")
    iota = lax.iota(jnp.int32, 16)

    def transpose(p, nr8):
        ev = 2 * iota
        od = 2 * iota + 1

        def r8body(r8, carry):
            for rr in range(8):
                col = jnp.full((16,), 0, jnp.int32) + (8 * r8 + rr)
                evens = plsc.load_gather(in_v.at[p], [ev, col])
                odds = plsc.load_gather(in_v.at[p], [od, col])
                w = plsc.bitcast(
                    plsc.pack(evens, odds,
                              format=plsc.PackFormat.INTERLEAVED),
                    jnp.int32)
                tout_v[p, r8, pl.ds(rr * 16, 16)] = w
            return carry
        lax.fori_loop(0, nr8, r8body, 0)

    def in_copy(m, p, sem):
        rt = m * NW + wid
        return pltpu.make_async_copy(
            tT_hbm.at[:, pl.ds(rt * CW, CW)], in_v.at[p], sem)

    def out_copy(m, p, sem):
        rt = m * NW + wid
        return pltpu.make_async_copy(
            tout_v.at[p], out_hbm.at[pl.ds(rt * (CW // 8), CW // 8)], sem)

    def valid(m):
        return (m * NW + wid) < VT

    M = 2 * ((VT + 2 * NW - 1) // (2 * NW))

    @pl.when(valid(0))
    def _():
        in_copy(0, 0, isem0).start()

    @pl.when(valid(1))
    def _():
        in_copy(1, 1, isem1).start()

    def pair(j2, carry):
        for p, isem, osem in ((0, isem0, osem0), (1, isem1, osem1)):
            m = 2 * j2 + p

            @pl.when(valid(m))
            def _():
                in_copy(m, p, isem).wait()

            @pl.when((m >= 2) & valid(m - 2))
            def _():
                out_copy(m - 2, p, osem).wait()

            @pl.when(valid(m))
            def _():
                transpose(p, CW // 8)

            @pl.when(valid(m + 2))
            def _():
                in_copy(m + 2, p, isem).start()

            @pl.when(valid(m))
            def _():
                out_copy(m, p, osem).start()
        return carry

    lax.fori_loop(0, M // 2, pair, 0)

    @pl.when(valid(M - 2))
    def _():
        out_copy(M - 2, 0, osem0).wait()

    @pl.when(valid(M - 1))
    def _():
        out_copy(M - 1, 1, osem1).wait()

    @pl.when(wid == 0)
    def _():
        pltpu.sync_copy(tail_hbm, in_v.at[0, :, pl.ds(0, BB)])
        transpose(0, 8)
        pltpu.sync_copy(tout_v.at[0, pl.ds(0, 8)],
                        out_hbm.at[pl.ds(VT * (CW // 8), 8)])


def _gather_body(idxT_hbm, table_hbm, out_hbm, idx_v, rows_v, tout_v,
                 gsem0, gsem1, ssem0, ssem1):
    wid = lax.axis_index("s") * NC + lax.axis_index("c")
    pltpu.sync_copy(idxT_hbm.at[:, pl.ds(wid * BB, BB)], idx_v)
    iota = lax.iota(jnp.int32, 16)

    def gather(s, p, sem):
        return pltpu.make_async_copy(
            table_hbm.at[idx_v.at[s]], rows_v.at[p], sem)

    def store(s, p, sem):
        return pltpu.make_async_copy(
            tout_v.at[p], out_hbm.at[s, :, pl.ds(wid * 1024, 1024)], sem)

    def transpose(p):
        himask = jnp.full((16,), -65536, jnp.int32)
        for bk in range(8):
            rid = iota + bk * 16
            for fg in range(4):
                ws = [
                    plsc.load_gather(
                        rows_v.at[p],
                        [rid, jnp.full((16,), fg * 4 + j, jnp.int32)])
                    for j in range(4)
                ]
                for j in range(4):
                    even = plsc.bitcast(ws[j] << 16, jnp.float32)
                    odd = plsc.bitcast(ws[j] & himask, jnp.float32)
                    tout_v[p, fg, pl.ds((2 * j) * BB + bk * 16, 16)] = even
                    tout_v[p, fg, pl.ds((2 * j + 1) * BB + bk * 16, 16)] = odd

    gather(0, 0, gsem0).start()
    gather(1, 1, gsem1).start()

    def pair(j2, carry):
        for p, gsem, ssem in ((0, gsem0, ssem0), (1, gsem1, ssem1)):
            s = 2 * j2 + p
            gather(s, p, gsem).wait()

            @pl.when(j2 >= 1)
            def _():
                store(s - 2, p, ssem).wait()

            transpose(p)

            @pl.when(j2 < S // 2 - 1)
            def _():
                gather(s + 2, p, gsem).start()

            store(s, p, ssem).start()
        return carry

    lax.fori_loop(0, S // 2, pair, 0)
    store(S - 2, 0, ssem0).wait()
    store(S - 1, 1, ssem1).wait()


def kernel(inputs, embedding):
    bt, s = inputs.shape
    v, d = embedding.shape
    assert (bt, s, v, d) == (NW * BB, S, V, 32)

    mesh = plsc.VectorSubcoreMesh(core_axis_name="c", subcore_axis_name="s")

    tT = embedding.T
    tail = jnp.pad(tT[:, VT * CW:], ((0, 0), (0, BB - VTAIL)))
    k1 = pl.kernel(
        _relayout_body,
        out_type=jax.ShapeDtypeStruct((V * (d // 2) // BB, BB), jnp.int32),
        mesh=mesh,
        scratch_types=[
            pltpu.VMEM((2, d, CW), jnp.float32),
            pltpu.VMEM((2, CW // 8, BB), jnp.int32),
            pltpu.SemaphoreType.DMA,
            pltpu.SemaphoreType.DMA,
            pltpu.SemaphoreType.DMA,
            pltpu.SemaphoreType.DMA,
        ],
        compiler_params=pltpu.CompilerParams(
            use_tc_tiling_on_sc=True, needs_layout_passes=False),
    )
    table_rm = k1(tT, tail).reshape(v, d // 2)

    idxT = inputs.T.astype(jnp.int32)
    k2 = pl.kernel(
        _gather_body,
        out_type=jax.ShapeDtypeStruct((S, 4, 8 * BB * NW), jnp.float32),
        mesh=mesh,
        scratch_types=[
            pltpu.VMEM((S, BB), jnp.int32),
            pltpu.VMEM((2, BB, d // 2), jnp.int32),
            pltpu.VMEM((2, 4, 8 * BB), jnp.float32),
            pltpu.SemaphoreType.DMA,
            pltpu.SemaphoreType.DMA,
            pltpu.SemaphoreType.DMA,
            pltpu.SemaphoreType.DMA,
        ],
        compiler_params=pltpu.CompilerParams(
            use_tc_tiling_on_sc=False, needs_layout_passes=False),
    )
    out5 = k2(idxT, table_rm)
    return (out5.reshape(S, 4, NW, 8, BB)
                .transpose(2, 4, 0, 1, 3)
                .reshape(bt, s, d))
